# baseline (speedup 1.0000x reference)
import functools

import jax
import jax.numpy as jnp
from jax.experimental import pallas as pl
from jax.experimental.pallas import tpu as pltpu

N_TOKENS = 16384
D_MODEL = 2048
N_EXP = 64
N_TOPICS = 4
BN = 1024
NSPLIT = 2
BT = BN * NSPLIT


def _routing_block(s, idx_ref, row0):
    iota = jax.lax.broadcasted_iota(jnp.int32, s.shape, 1)
    m1 = jnp.max(s, axis=-1, keepdims=True)
    i1 = jnp.min(jnp.where(s == m1, iota, N_EXP), axis=-1, keepdims=True)
    s2 = jnp.where(iota == i1, -jnp.inf, s)
    m2 = jnp.max(s2, axis=-1, keepdims=True)
    i2 = jnp.min(jnp.where(s2 == m2, iota, N_EXP), axis=-1, keepdims=True)
    lane2 = jax.lax.broadcasted_iota(jnp.int32, (s.shape[0], 2), 1)
    idx_ref[pl.ds(row0, s.shape[0]), :] = jnp.where(lane2 == 0, i1, i2)
    hit = ((iota == i1) | (iota == i2)).astype(jnp.float32)
    return jnp.sum(hit, axis=0, keepdims=True)


def _fused_body(*refs):
    u_refs = refs[:NSPLIT]
    wg_ref, wh_ref, bh_ref, head_ref, idx_ref, aux_ref, dens_ref, prox_ref = \
        refs[NSPLIT:]
    step = pl.program_id(0)
    nsteps = pl.num_programs(0)

    @pl.when(step == 0)
    def _init():
        dens_ref[...] = jnp.zeros_like(dens_ref)
        prox_ref[...] = jnp.zeros_like(prox_ref)

    dens = jnp.zeros((1, N_EXP), jnp.float32)
    prox = jnp.zeros((1, N_EXP), jnp.float32)
    for j in range(NSPLIT):
        logits = jnp.dot(u_refs[j][...], wg_ref[...],
                         preferred_element_type=jnp.float32)
        m = jnp.max(logits, axis=-1, keepdims=True)
        ex = jnp.exp(logits - m)
        s = ex / jnp.sum(ex, axis=-1, keepdims=True)
        head_ref[pl.ds(j * BN, BN), :] = (
            jnp.dot(s, wh_ref[...], preferred_element_type=jnp.float32)
            + bh_ref[...])
        dens += _routing_block(s, idx_ref, j * BN)
        prox += jnp.sum(s, axis=0, keepdims=True)
    dens_ref[...] += dens
    prox_ref[...] += prox

    @pl.when(step == nsteps - 1)
    def _finish():
        n = jnp.float32(N_TOKENS)
        aux_ref[...] = (jnp.float32(N_EXP)
                        * jnp.sum(dens_ref[...] * prox_ref[...],
                                  axis=1, keepdims=True) / (n * n))


@jax.jit
def _fused(u, W_g, W_h, b_h2):
    grid = (N_TOKENS // BT,)
    u_specs = [
        pl.BlockSpec((BN, D_MODEL), functools.partial(
            lambda i, jj: (i * NSPLIT + jj, 0), jj=j))
        for j in range(NSPLIT)
    ]
    head, idx, aux = pl.pallas_call(
        _fused_body,
        grid=grid,
        in_specs=u_specs + [
            pl.BlockSpec((D_MODEL, N_EXP), lambda i: (0, 0)),
            pl.BlockSpec((N_EXP, N_TOPICS), lambda i: (0, 0)),
            pl.BlockSpec((1, N_TOPICS), lambda i: (0, 0)),
        ],
        out_specs=[
            pl.BlockSpec((BT, N_TOPICS), lambda i: (i, 0)),
            pl.BlockSpec((BT, 2), lambda i: (i, 0)),
            pl.BlockSpec((1, 1), lambda i: (0, 0)),
        ],
        out_shape=[
            jax.ShapeDtypeStruct((N_TOKENS, N_TOPICS), jnp.float32),
            jax.ShapeDtypeStruct((N_TOKENS, 2), jnp.int32),
            jax.ShapeDtypeStruct((1, 1), jnp.float32),
        ],
        scratch_shapes=[
            pltpu.VMEM((1, N_EXP), jnp.float32),
            pltpu.VMEM((1, N_EXP), jnp.float32),
        ],
    )(*([u] * NSPLIT), W_g, W_h, b_h2)
    return head, idx, aux


def kernel(u, W_g, W_h, b_h):
    head, idx, aux = _fused(u, W_g, W_h, b_h.reshape(1, N_TOPICS))
    return (head, aux.reshape(()), idx)
